# Initial kernel scaffold; baseline (speedup 1.0000x reference)
import functools

import jax
import jax.numpy as jnp
from jax import lax
from jax.experimental import pallas as pl
from jax.experimental.pallas import tpu as pltpu
from jax.experimental.pallas import tpu_sc as plsc

N = 100000
D = 128
NSEG = 16
NC = 2
NS = 16
NW = NC * NS
RPT = 3128
CHUNK = 512
LANES = 16
DBLK = D // LANES

_mesh = plsc.VectorSubcoreMesh(
    core_axis_name="c", subcore_axis_name="s", num_cores=NC, num_subcores=NS
)


@functools.partial(
    pl.kernel,
    out_type=jax.ShapeDtypeStruct((NW, NSEG, D), jnp.float32),
    mesh=_mesh,
    scratch_types=[
        pltpu.VMEM((RPT,), jnp.int32),
        pltpu.VMEM((CHUNK, D), jnp.float32),
        pltpu.VMEM((NSEG, D), jnp.float32),
        pltpu.VMEM((LANES,), jnp.int32),
    ],
)
def _sc_segmax(x_hbm, b_hbm, out_hbm, bv, buf, accv, lbv):
    wid = lax.axis_index("s") * NC + lax.axis_index("c")
    r0 = jnp.minimum(wid * RPT, N - RPT)

    pltpu.sync_copy(b_hbm.at[pl.ds(r0, RPT)], bv)

    seg_ids = lax.iota(jnp.int32, LANES)
    pos = jnp.zeros((LANES,), jnp.int32)
    step = 2048
    while step >= 1:
        cand = pos + step
        idx = jnp.minimum(cand, RPT) - 1
        vals = plsc.load_gather(bv, [idx])
        take = (cand <= RPT) & (vals < seg_ids)
        pos = jnp.where(take, cand, pos)
        step //= 2
    lbv[...] = pos

    neg_inf = jnp.full((LANES,), -jnp.inf, jnp.float32)
    for s in range(NSEG):
        lo = lbv[s]
        hi = lbv[s + 1] if s < NSEG - 1 else jnp.int32(RPT)
        cnt = hi - lo
        nch = lax.shift_right_logical(cnt + (CHUNK - 1), 9)

        def chunk_body(k, accs, lo=lo, cnt=cnt):
            g = r0 + lo + k * CHUNK
            gc = jnp.minimum(g, N - CHUNK)
            shift = g - gc
            pltpu.sync_copy(x_hbm.at[pl.ds(gc, CHUNK)], buf)
            rows = jnp.minimum(jnp.int32(CHUNK), cnt - k * CHUNK)

            def row_body(i, a):
                r = shift + i
                return tuple(
                    jnp.maximum(a[j], buf[r, pl.ds(j * LANES, LANES)])
                    for j in range(DBLK)
                )

            return lax.fori_loop(0, rows, row_body, accs)

        accs = lax.fori_loop(0, nch, chunk_body, (neg_inf,) * DBLK)
        for j in range(DBLK):
            accv[s, pl.ds(j * LANES, LANES)] = accs[j]

    pltpu.sync_copy(accv, out_hbm.at[wid])


def _fold_body(p_ref, o_ref):
    o_ref[...] = jnp.max(p_ref[...], axis=0)


@jax.jit
def kernel(x, pos, batch):
    batch32 = batch.astype(jnp.int32)
    partial = _sc_segmax(x, batch32)
    x_max = pl.pallas_call(
        _fold_body,
        out_shape=jax.ShapeDtypeStruct((NSEG, D), jnp.float32),
    )(partial)
    pos_out = jnp.zeros((NSEG, 3), dtype=pos.dtype)
    batch_out = jnp.arange(NSEG, dtype=batch.dtype)
    return (x_max, pos_out, batch_out)


# speedup vs baseline: 5.5987x; 5.5987x over previous
import functools

import jax
import jax.numpy as jnp
from jax import lax
from jax.experimental import pallas as pl
from jax.experimental.pallas import tpu as pltpu
from jax.experimental.pallas import tpu_sc as plsc

N = 100000
D = 128
NSEG = 16
NC = 2
NS = 16
NW = NC * NS
RPT = 3128
CHUNK = 512
CSTEP = CHUNK - 8
LANES = 16
DBLK = D // LANES

_mesh = plsc.VectorSubcoreMesh(
    core_axis_name="c", subcore_axis_name="s", num_cores=NC, num_subcores=NS
)


@functools.partial(
    pl.kernel,
    out_type=jax.ShapeDtypeStruct((NW, NSEG, D), jnp.float32),
    mesh=_mesh,
    scratch_types=[
        pltpu.VMEM((RPT,), jnp.int32),
        pltpu.VMEM((CHUNK, D), jnp.float32),
        pltpu.VMEM((NSEG, D), jnp.float32),
    ],
    compiler_params=pltpu.CompilerParams(needs_layout_passes=False),
)
def _sc_segmax(x_hbm, b_hbm, out_hbm, bv, buf, accv):
    wid = lax.axis_index("s") * NC + lax.axis_index("c")
    r0 = pl.multiple_of(jnp.minimum(wid * RPT, N - RPT), 8)

    pltpu.sync_copy(b_hbm.at[pl.ds(r0, RPT)], bv)

    seg_ids = lax.iota(jnp.int32, LANES)
    pos = jnp.zeros((LANES,), jnp.int32)
    step = 2048
    while step >= 1:
        cand = pos + step
        idx = jnp.minimum(cand, RPT) - 1
        vals = plsc.load_gather(bv, [idx])
        take = (cand <= RPT) & (vals < seg_ids)
        pos = jnp.where(take, cand, pos)
        step //= 2

    neg_inf = jnp.full((LANES,), -jnp.inf, jnp.float32)
    for s in range(NSEG):
        lo = pos[s]
        hi = pos[s + 1] if s < NSEG - 1 else jnp.int32(RPT)
        cnt = hi - lo
        nch = lax.div(cnt + (CSTEP - 1), jnp.int32(CSTEP))

        def chunk_body(k, accs, lo=lo, cnt=cnt):
            g = r0 + lo + k * CSTEP
            ga = lax.bitwise_and(g, jnp.int32(~7))
            gc = pl.multiple_of(jnp.minimum(ga, N - CHUNK), 8)
            shift = g - gc
            pltpu.sync_copy(x_hbm.at[pl.ds(gc, CHUNK)], buf)
            rows = jnp.minimum(jnp.int32(CSTEP), cnt - k * CSTEP)

            def row_body(i, a):
                r = shift + i
                return tuple(
                    jnp.maximum(a[j], buf[r, pl.ds(j * LANES, LANES)])
                    for j in range(DBLK)
                )

            return lax.fori_loop(0, rows, row_body, accs)

        accs = lax.fori_loop(0, nch, chunk_body, (neg_inf,) * DBLK)
        for j in range(DBLK):
            accv[s, pl.ds(j * LANES, LANES)] = accs[j]

    pltpu.sync_copy(accv, out_hbm.at[wid])


def _fold_body(p_ref, o_ref):
    o_ref[...] = jnp.max(p_ref[...], axis=0)


@jax.jit
def kernel(x, pos, batch):
    batch32 = batch.astype(jnp.int32)
    partial = _sc_segmax(x, batch32)
    x_max = pl.pallas_call(
        _fold_body,
        out_shape=jax.ShapeDtypeStruct((NSEG, D), jnp.float32),
    )(partial)
    pos_out = jnp.zeros((NSEG, 3), dtype=pos.dtype)
    batch_out = jnp.arange(NSEG, dtype=batch.dtype)
    return (x_max, pos_out, batch_out)


# speedup vs baseline: 5.8129x; 1.0383x over previous
import functools

import jax
import jax.numpy as jnp
from jax import lax
from jax.experimental import pallas as pl
from jax.experimental.pallas import tpu as pltpu
from jax.experimental.pallas import tpu_sc as plsc

N = 100000
D = 128
NSEG = 16
NC = 2
NS = 16
NW = NC * NS
CHUNK = 128
NCHUNK = 26
RPT = CHUNK * NCHUNK
STRIDE = 3128
LANES = 16
DBLK = D // LANES

_mesh = plsc.VectorSubcoreMesh(
    core_axis_name="c", subcore_axis_name="s", num_cores=NC, num_subcores=NS
)


@functools.partial(
    pl.kernel,
    out_type=jax.ShapeDtypeStruct((NW, NSEG, D), jnp.float32),
    mesh=_mesh,
    scratch_types=[
        pltpu.VMEM((RPT,), jnp.int32),
        pltpu.VMEM((CHUNK, D), jnp.float32),
        pltpu.VMEM((CHUNK, D), jnp.float32),
        pltpu.VMEM((NSEG, D), jnp.float32),
        pltpu.SemaphoreType.DMA,
        pltpu.SemaphoreType.DMA,
    ],
    compiler_params=pltpu.CompilerParams(needs_layout_passes=False),
)
def _sc_segmax(x_hbm, b_hbm, out_hbm, bv, buf0, buf1, accv, sem0, sem1):
    wid = lax.axis_index("s") * NC + lax.axis_index("c")
    r0 = pl.multiple_of(jnp.minimum(wid * STRIDE, N - RPT), 8)

    pltpu.async_copy(x_hbm.at[pl.ds(r0, CHUNK)], buf0, sem0)
    pltpu.sync_copy(b_hbm.at[pl.ds(r0, RPT)], bv)

    seg_ids = lax.iota(jnp.int32, LANES)
    pos = jnp.zeros((LANES,), jnp.int32)
    step = 2048
    while step >= 1:
        cand = pos + step
        idx = jnp.minimum(cand, RPT) - 1
        vals = plsc.load_gather(bv, [idx])
        take = (cand <= RPT) & (vals < seg_ids)
        pos = jnp.where(take, cand, pos)
        step //= 2

    neg_inf = jnp.full((LANES,), -jnp.inf, jnp.float32)
    for s in range(NSEG):
        for j in range(DBLK):
            accv[s, pl.ds(j * LANES, LANES)] = neg_inf

    def process(c, mybuf, nxtbuf, mysem, nxtsem):
        pltpu.make_async_copy(x_hbm.at[pl.ds(0, CHUNK)], mybuf, mysem).wait()

        @pl.when(c + 1 < NCHUNK)
        def _():
            g = pl.multiple_of(r0 + (c + 1) * CHUNK, 8)
            pltpu.async_copy(x_hbm.at[pl.ds(g, CHUNK)], nxtbuf, nxtsem)

        c0 = c * CHUNK
        sfirst = plsc.all_reduce_population_count(pos <= c0)[0] - 1
        slast = plsc.all_reduce_population_count(pos <= c0 + (CHUNK - 1))[0] - 1

        @pl.when(sfirst == slast)
        def _():
            def row_body(i, a):
                return tuple(
                    jnp.maximum(a[j], mybuf[i, pl.ds(j * LANES, LANES)])
                    for j in range(DBLK)
                )

            accs = lax.fori_loop(
                0, CHUNK, row_body, (neg_inf,) * DBLK, unroll=4
            )
            for j in range(DBLK):
                sl = pl.ds(j * LANES, LANES)
                accv[sfirst, sl] = jnp.maximum(accv[sfirst, sl], accs[j])

        @pl.when(sfirst != slast)
        def _():
            def grp_body(gi, carry):
                b16 = bv[pl.ds(c0 + gi * LANES, LANES)]
                for lane in range(LANES):
                    seg = b16[lane]
                    r = gi * LANES + lane
                    for j in range(DBLK):
                        sl = pl.ds(j * LANES, LANES)
                        accv[seg, sl] = jnp.maximum(accv[seg, sl], mybuf[r, sl])
                return carry

            lax.fori_loop(0, CHUNK // LANES, grp_body, 0)

    def pair_body(t, carry):
        process(2 * t, buf0, buf1, sem0, sem1)
        process(2 * t + 1, buf1, buf0, sem1, sem0)
        return carry

    lax.fori_loop(0, NCHUNK // 2, pair_body, 0)

    pltpu.sync_copy(accv, out_hbm.at[wid])


def _fold_body(p_ref, o_ref):
    o_ref[...] = jnp.max(p_ref[...], axis=0)


@jax.jit
def kernel(x, pos, batch):
    batch32 = batch.astype(jnp.int32)
    partial = _sc_segmax(x, batch32)
    x_max = pl.pallas_call(
        _fold_body,
        out_shape=jax.ShapeDtypeStruct((NSEG, D), jnp.float32),
    )(partial)
    pos_out = jnp.zeros((NSEG, 3), dtype=pos.dtype)
    batch_out = jnp.arange(NSEG, dtype=batch.dtype)
    return (x_max, pos_out, batch_out)


# speedup vs baseline: 6.3124x; 1.0859x over previous
import functools

import jax
import jax.numpy as jnp
from jax import lax
from jax.experimental import pallas as pl
from jax.experimental.pallas import tpu as pltpu
from jax.experimental.pallas import tpu_sc as plsc

N = 100000
D = 128
NSEG = 16
NC = 2
NS = 16
NW = NC * NS
CHUNK = 112
NCHUNK = 28
NBUF = 7
RPT = CHUNK * NCHUNK
STRIDE = 3128
LANES = 16
DBLK = D // LANES

_mesh = plsc.VectorSubcoreMesh(
    core_axis_name="c", subcore_axis_name="s", num_cores=NC, num_subcores=NS
)


@functools.partial(
    pl.kernel,
    out_type=jax.ShapeDtypeStruct((NW, NSEG, D), jnp.float32),
    mesh=_mesh,
    scratch_types=[
        pltpu.VMEM((RPT,), jnp.int32),
        [pltpu.VMEM((CHUNK, D), jnp.float32) for _ in range(NBUF)],
        pltpu.VMEM((NSEG, D), jnp.float32),
        [pltpu.SemaphoreType.DMA for _ in range(NBUF)],
    ],
    compiler_params=pltpu.CompilerParams(needs_layout_passes=False),
)
def _sc_segmax(x_hbm, b_hbm, out_hbm, bv, bufs, accv, sems):
    wid = lax.axis_index("s") * NC + lax.axis_index("c")
    r0 = pl.multiple_of(jnp.minimum(wid * STRIDE, N - RPT), 8)

    for c in range(NBUF - 1):
        pltpu.async_copy(
            x_hbm.at[pl.ds(r0 + c * CHUNK, CHUNK)], bufs[c], sems[c]
        )
    pltpu.sync_copy(b_hbm.at[pl.ds(r0, RPT)], bv)

    seg_ids = lax.iota(jnp.int32, LANES)
    pos = jnp.zeros((LANES,), jnp.int32)
    step = 2048
    while step >= 1:
        cand = pos + step
        idx = jnp.minimum(cand, RPT) - 1
        vals = plsc.load_gather(bv, [idx])
        take = (cand <= RPT) & (vals < seg_ids)
        pos = jnp.where(take, cand, pos)
        step //= 2

    neg_inf = jnp.full((LANES,), -jnp.inf, jnp.float32)
    for s in range(NSEG):
        for j in range(DBLK):
            accv[s, pl.ds(j * LANES, LANES)] = neg_inf

    def process(c, mybuf, nxtbuf, mysem, nxtsem):
        pltpu.make_async_copy(x_hbm.at[pl.ds(0, CHUNK)], mybuf, mysem).wait()

        @pl.when(c + (NBUF - 1) < NCHUNK)
        def _():
            g = pl.multiple_of(r0 + (c + (NBUF - 1)) * CHUNK, 8)
            pltpu.async_copy(x_hbm.at[pl.ds(g, CHUNK)], nxtbuf, nxtsem)

        c0 = c * CHUNK
        sfirst = plsc.all_reduce_population_count(pos <= c0)[0] - 1
        slast = plsc.all_reduce_population_count(pos <= c0 + (CHUNK - 1))[0] - 1

        @pl.when(sfirst == slast)
        def _():
            def row_body(i, a):
                return tuple(
                    jnp.maximum(a[j], mybuf[i, pl.ds(j * LANES, LANES)])
                    for j in range(DBLK)
                )

            accs = lax.fori_loop(
                0, CHUNK, row_body, (neg_inf,) * DBLK, unroll=4
            )
            for j in range(DBLK):
                sl = pl.ds(j * LANES, LANES)
                accv[sfirst, sl] = jnp.maximum(accv[sfirst, sl], accs[j])

        @pl.when(sfirst != slast)
        def _():
            def grp_body(gi, carry):
                b16 = bv[pl.ds(c0 + gi * LANES, LANES)]
                for lane in range(LANES):
                    seg = b16[lane]
                    r = gi * LANES + lane
                    for j in range(DBLK):
                        sl = pl.ds(j * LANES, LANES)
                        accv[seg, sl] = jnp.maximum(accv[seg, sl], mybuf[r, sl])
                return carry

            lax.fori_loop(0, CHUNK // LANES, grp_body, 0)

    def ring_body(t, carry):
        for b in range(NBUF):
            c = NBUF * t + b
            nxt = (b + NBUF - 1) % NBUF
            process(c, bufs[b], bufs[nxt], sems[b], sems[nxt])
        return carry

    lax.fori_loop(0, NCHUNK // NBUF, ring_body, 0)

    pltpu.sync_copy(accv, out_hbm.at[wid])


def _fold_body(p_ref, o_ref):
    o_ref[...] = jnp.max(p_ref[...], axis=0)


@jax.jit
def kernel(x, pos, batch):
    batch32 = batch.astype(jnp.int32)
    partial = _sc_segmax(x, batch32)
    x_max = pl.pallas_call(
        _fold_body,
        out_shape=jax.ShapeDtypeStruct((NSEG, D), jnp.float32),
    )(partial)
    pos_out = jnp.zeros((NSEG, 3), dtype=pos.dtype)
    batch_out = jnp.arange(NSEG, dtype=batch.dtype)
    return (x_max, pos_out, batch_out)
